# Initial kernel scaffold; baseline (speedup 1.0000x reference)
import jax
import jax.numpy as jnp
from jax.experimental import pallas as pl


def kernel(obs_features, edge_indices, edge_features, feat_features, params):
    raise NotImplementedError("write your pallas kernel here")


# speedup vs baseline: 1.0551x; 1.0551x over previous
import functools
import jax
import jax.numpy as jnp
from jax import lax
from jax.experimental import pallas as pl

_EPS = 1e-5


def _mlp_in(x, p):
    z = jnp.dot(x, p["W1"], preferred_element_type=jnp.float32) + p["b1"]
    zn = (z - jnp.mean(z, axis=0, keepdims=True)) / jnp.sqrt(
        jnp.var(z, axis=0, keepdims=True) + _EPS)
    return jax.nn.relu(
        jnp.dot(zn, p["W2"], preferred_element_type=jnp.float32) + p["b2"])


def _wmlp_in(x, p, w, n_edges):
    z = jnp.dot(x, p["W1"], preferred_element_type=jnp.float32) + p["b1"]
    m = jnp.sum(w * z, axis=0, keepdims=True) / n_edges
    v = jnp.sum(w * z * z, axis=0, keepdims=True) / n_edges - m * m
    zn = (z - m) / jnp.sqrt(v + _EPS)
    return jax.nn.relu(
        jnp.dot(zn, p["W2"], preferred_element_type=jnp.float32) + p["b2"])


def _node_pre_body(obs_ref, feat_ref, ef_ref, cntd_ref, cnts_ref,
                   po_W1, po_b1, po_W2, po_b2,
                   pf_W1, pf_b1, pf_W2, pf_b2,
                   r_W1, r_b1, r_W2, r_b2,
                   e_W1, e_b1, e_W2, e_b2,
                   eW_ref, eb_ref, f_W1top, f_W1bot,
                   P_ref, Q_ref, feat0_ref, t_ref, u_ref):
    n_edges = jnp.float32(ef_ref.shape[0] * ef_ref.shape[1])
    obs0 = _mlp_in(obs_ref[...], {"W1": po_W1[...], "b1": po_b1[...],
                                  "W2": po_W2[...], "b2": po_b2[...]})
    feat0 = _mlp_in(feat_ref[...], {"W1": pf_W1[...], "b1": pf_b1[...],
                                    "W2": pf_W2[...], "b2": pf_b2[...]})
    feat0_ref[...] = feat0
    rp = {"W1": r_W1[...], "b1": r_b1[...], "W2": r_W2[...], "b2": r_b2[...]}
    A = _wmlp_in(feat0, rp, cntd_ref[...], n_edges)
    C = _wmlp_in(obs0, rp, cnts_ref[...], n_edges)
    P_ref[...] = jnp.dot(A, f_W1top[...], preferred_element_type=jnp.float32)
    Q_ref[...] = jnp.dot(C, f_W1bot[...], preferred_element_type=jnp.float32)
    ef = ef_ref[...]
    mu = jnp.mean(ef)
    var = jnp.mean(ef * ef) - mu * mu
    a = jnp.dot(eW_ref[...], e_W1[...], preferred_element_type=jnp.float32)
    g = a / jnp.sqrt(var * a * a + _EPS)
    u = jnp.dot(g, e_W2[...], preferred_element_type=jnp.float32)
    t_ref[...] = ef - mu
    u_ref[...] = u


def _node_pre(obs_f, feat_f, ef, cntd, cnts, params):
    N = obs_f.shape[0]
    E = ef.shape[0]
    po, pf = params["proj_obs"], params["proj_feat"]
    cp = params["conv_o_to_f"]
    rp, ep, fp = cp["right"], cp["edge"], cp["final"]
    eW = params["edge_W"]
    eb = params["edge_b"].reshape(1, -1)
    out_shapes = (
        jax.ShapeDtypeStruct((N, 64), jnp.float32),
        jax.ShapeDtypeStruct((N, 64), jnp.float32),
        jax.ShapeDtypeStruct((N, 64), jnp.float32),
        jax.ShapeDtypeStruct((1, E), jnp.float32),
        jax.ShapeDtypeStruct((1, 64), jnp.float32),
    )
    args = (obs_f, feat_f, ef.reshape(1, E), cntd.reshape(N, 1), cnts.reshape(N, 1),
            po["W1"], po["b1"].reshape(1, -1), po["W2"], po["b2"].reshape(1, -1),
            pf["W1"], pf["b1"].reshape(1, -1), pf["W2"], pf["b2"].reshape(1, -1),
            rp["W1"], rp["b1"].reshape(1, -1), rp["W2"], rp["b2"].reshape(1, -1),
            ep["W1"], ep["b1"].reshape(1, -1), ep["W2"], ep["b2"].reshape(1, -1),
            eW, eb, fp["W1"][0:64], fp["W1"][128:192])
    return pl.pallas_call(_node_pre_body, out_shape=out_shapes)(*args)


def _edge_R_body(t_ref, uc_ref, b2ec_ref, W1mT_ref, b1fc_ref, rmT_ref):
    rel = jax.nn.relu(uc_ref[...] * t_ref[...] + b2ec_ref[...])
    rmT_ref[...] = jnp.dot(W1mT_ref[...], rel,
                           preferred_element_type=jnp.float32) + b1fc_ref[...]


def _edge_R(t, u, b2e, W1m, b1f, blk=16000):
    E = t.shape[1]
    grid = (E // blk,)
    return pl.pallas_call(
        _edge_R_body,
        grid=grid,
        in_specs=[
            pl.BlockSpec((1, blk), lambda i: (0, i)),
            pl.BlockSpec((64, 1), lambda i: (0, 0)),
            pl.BlockSpec((64, 1), lambda i: (0, 0)),
            pl.BlockSpec((64, 64), lambda i: (0, 0)),
            pl.BlockSpec((64, 1), lambda i: (0, 0)),
        ],
        out_specs=pl.BlockSpec((64, blk), lambda i: (0, i)),
        out_shape=jax.ShapeDtypeStruct((64, E), jnp.float32),
    )(t, u.reshape(64, 1), b2e.reshape(64, 1), W1m.T, b1f.reshape(64, 1))


def _final_body(S_ref, cntd_ref, feat0_ref,
                o_W1, o_b1, o_W2, o_b2, yW1, yb1, yW2, yb2, out_ref):
    agg = S_ref[...] / jnp.maximum(cntd_ref[...], 1.0)
    x = jnp.concatenate([agg, feat0_ref[...]], axis=1)
    feat1 = _mlp_in(x, {"W1": o_W1[...], "b1": o_b1[...],
                        "W2": o_W2[...], "b2": o_b2[...]})
    h = jnp.dot(feat1, yW1[...], preferred_element_type=jnp.float32) + yb1[...]
    h = jnp.where(h > 0, h, 0.01 * h)
    out_ref[...] = jnp.dot(h, yW2[...],
                           preferred_element_type=jnp.float32) + yb2[...]


def _final(S, cntd, feat0, params):
    N = S.shape[0]
    op = params["conv_o_to_f"]["output"]
    return pl.pallas_call(
        _final_body,
        out_shape=jax.ShapeDtypeStruct((N, 1), jnp.float32),
    )(S, cntd.reshape(N, 1), feat0,
      op["W1"], op["b1"].reshape(1, -1), op["W2"], op["b2"].reshape(1, -1),
      params["Y_W1"], params["Y_b1"].reshape(1, -1),
      params["Y_W2"], params["Y_b2"].reshape(1, -1))


def kernel(obs_features, edge_indices, edge_features, feat_features, params):
    N = obs_features.shape[0]
    E = edge_features.shape[0]
    src = edge_indices[0]
    dst = edge_indices[1]
    ones = jnp.ones((E,), jnp.float32)
    cntd = jax.ops.segment_sum(ones, dst, num_segments=N)
    cnts = jax.ops.segment_sum(ones, src, num_segments=N)

    P, Q, feat0, t, u = _node_pre(obs_features, feat_features, edge_features,
                                  cntd, cnts, params)
    cp = params["conv_o_to_f"]
    fp = cp["final"]
    b2e = cp["edge"]["b2"].reshape(1, -1)
    RmT = _edge_R(t, u, b2e, fp["W1"][64:128], fp["b1"].reshape(1, -1))

    Y = jnp.take(P, dst, axis=0) + jnp.take(Q, src, axis=0) + RmT.T
    m = jnp.mean(Y, axis=0, keepdims=True)
    v = jnp.var(Y, axis=0, keepdims=True)
    Yn = (Y - m) / jnp.sqrt(v + _EPS)
    msg = jax.nn.relu(Yn @ fp["W2"] + fp["b2"])
    S = jax.ops.segment_sum(msg, dst, num_segments=N)

    return _final(S, cntd, feat0, params)
